# Initial kernel scaffold; baseline (speedup 1.0000x reference)
import jax
import jax.numpy as jnp
from jax.experimental import pallas as pl


def kernel(x, edge_index, batch, W1, b1, gamma, beta, W2, b2, lin1_w, lin1_b, lin2_w, lin2_b):
    raise NotImplementedError("write your pallas kernel here")


# speedup vs baseline: 21.7941x; 21.7941x over previous
import functools

import jax
import jax.numpy as jnp
from jax import lax
from jax.experimental import pallas as pl
from jax.experimental.pallas import tpu as pltpu
from jax.experimental.pallas import tpu_sc as plsc

N_NODES = 100000
H = 16
G = 512
L = 7
N_CH = 782
N_P = N_CH * 128
N_PAD = N_P - N_NODES
E_EDGES = 3200000
NW = 32
ECH_W = 784
E_W = ECH_W * 128
E_P = E_W * NW
KG = 8
EGRP = ECH_W // KG

_mesh = plsc.VectorSubcoreMesh(core_axis_name="c", subcore_axis_name="s")
_sc_params = pltpu.CompilerParams(use_tc_tiling_on_sc=False)


def _zero_rows(buf):
    @pl.loop(0, 128)
    def _(i):
        buf[i, :] = jnp.zeros((H,), jnp.float32)


@functools.partial(
    pl.kernel,
    mesh=_mesh,
    out_type=(
        jax.ShapeDtypeStruct((2, N_P, H), jnp.float32),
        jax.ShapeDtypeStruct((2, G, H), jnp.float32),
    ),
    scratch_types=[
        pltpu.VMEM((KG, 128), jnp.int32),
        pltpu.VMEM((KG, 128), jnp.int32),
        pltpu.VMEM((128, H), jnp.float32),
        pltpu.VMEM((1, 128), jnp.int32),
        pltpu.VMEM((128, H), jnp.float32),
        pltpu.VMEM_SHARED((N_P, H), jnp.float32),
        pltpu.VMEM_SHARED((G, H), jnp.float32),
        pltpu.SemaphoreType.DMA,
    ],
    compiler_params=_sc_params,
)
def _sc_agg_pool(h_hbm, src_hbm, dst_hbm, batch_hbm, agg_out, pool_out,
                 sidx, didx, rows, bidx, prow, acc, pacc, sem):
    cid = lax.axis_index("c")
    sid = lax.axis_index("s")
    wid = cid * 16 + sid

    _zero_rows(prow)

    @pl.loop(sid, N_CH, step=16)
    def _(i):
        pltpu.sync_copy(prow, acc.at[pl.ds(i * 128, 128)])

    @pl.when(sid < G // 128)
    def _():
        pltpu.sync_copy(prow, pacc.at[pl.ds(sid * 128, 128)])

    plsc.subcore_barrier()

    ebase = wid * ECH_W

    @pl.loop(0, EGRP)
    def _(g):
        gb = ebase + g * KG
        pltpu.sync_copy(src_hbm.at[pl.ds(gb, KG)], sidx)
        pltpu.sync_copy(dst_hbm.at[pl.ds(gb, KG)], didx)
        for j in range(KG):
            pltpu.async_copy(h_hbm.at[sidx.at[j]], rows, sem).wait()
            pltpu.sync_copy(rows, acc.at[didx.at[j]], add=True)

    @pl.loop(wid, N_CH, step=NW)
    def _(i):
        pltpu.sync_copy(batch_hbm.at[pl.ds(i, 1)], bidx)
        pltpu.sync_copy(h_hbm.at[pl.ds(i * 128, 128)], prow)
        pltpu.sync_copy(prow, pacc.at[bidx.at[0]], add=True)

    plsc.subcore_barrier()

    @pl.loop(sid, N_CH, step=16)
    def _(i):
        pltpu.sync_copy(acc.at[pl.ds(i * 128, 128)],
                        agg_out.at[cid, pl.ds(i * 128, 128)])

    @pl.when(sid == 0)
    def _():
        pltpu.sync_copy(pacc, pool_out.at[cid])


@functools.partial(
    pl.kernel,
    mesh=_mesh,
    out_type=jax.ShapeDtypeStruct((2, G, H), jnp.float32),
    scratch_types=[
        pltpu.VMEM((1, 128), jnp.int32),
        pltpu.VMEM((128, H), jnp.float32),
        pltpu.VMEM_SHARED((G, H), jnp.float32),
    ],
    compiler_params=_sc_params,
)
def _sc_pool(h_hbm, batch_hbm, pool_out, bidx, prow, pacc):
    cid = lax.axis_index("c")
    sid = lax.axis_index("s")
    wid = cid * 16 + sid

    _zero_rows(prow)

    @pl.when(sid < G // 128)
    def _():
        pltpu.sync_copy(prow, pacc.at[pl.ds(sid * 128, 128)])

    plsc.subcore_barrier()

    @pl.loop(wid, N_CH, step=NW)
    def _(i):
        pltpu.sync_copy(batch_hbm.at[pl.ds(i, 1)], bidx)
        pltpu.sync_copy(h_hbm.at[pl.ds(i * 128, 128)], prow)
        pltpu.sync_copy(prow, pacc.at[bidx.at[0]], add=True)

    plsc.subcore_barrier()

    @pl.when(sid == 0)
    def _():
        pltpu.sync_copy(pacc, pool_out.at[cid])


N_ROWS = N_P // 8
N_REAL = N_NODES // 8


def _tc_layer_body(h_ref, agg_ref, w1_ref, b1t_ref, b1s_ref, g_ref, bet_ref,
                   w2_ref, b2t_ref, o_ref):
    h = h_ref[...]
    mask = lax.broadcasted_iota(jnp.int32, (N_ROWS, 1), 0) < N_REAL
    agg = agg_ref[0] + agg_ref[1]
    z = h + jnp.where(mask, agg, 0.0)
    z = jnp.dot(z.astype(jnp.bfloat16), w1_ref[...].astype(jnp.bfloat16),
                preferred_element_type=jnp.float32) + b1t_ref[...]
    b1s = b1s_ref[...]

    def _fold(v):
        acc = v[:, 0:H]
        for i in range(1, 8):
            acc = acc + v[:, i * H:(i + 1) * H]
        return acc

    def _spread(v):
        return jnp.concatenate([v] * 8, axis=1)

    s = _fold(jnp.sum(z, axis=0, keepdims=True))
    mean16 = (s - N_PAD * b1s) / N_NODES
    d = z - _spread(mean16)
    ssq = _fold(jnp.sum(d * d, axis=0, keepdims=True))
    dp = b1s - mean16
    var16 = (ssq - N_PAD * dp * dp) / N_NODES
    zn = d / _spread(jnp.sqrt(var16 + 1e-5)) * _spread(g_ref[...])
    zn = zn + bet_ref[...]
    a = jnp.maximum(zn, 0.0)
    z2 = jnp.dot(a.astype(jnp.bfloat16), w2_ref[...].astype(jnp.bfloat16),
                 preferred_element_type=jnp.float32) + b2t_ref[...]
    o_ref[...] = jnp.where(mask, jnp.maximum(z2, 0.0), 0.0)


def _tc_layer(h2, aggp2, w1b, b1t, b1s, gamma, betat, w2b, b2t):
    return pl.pallas_call(
        _tc_layer_body,
        out_shape=jax.ShapeDtypeStruct((N_ROWS, 128), jnp.float32),
    )(h2, aggp2, w1b, b1t, b1s, gamma, betat, w2b, b2t)


def _tc_readout_body(p_ref, w1_ref, b1_ref, w2_ref, b2_ref, sig_ref, out_ref):
    p = p_ref[...]
    ps = p[:, 0] + p[:, 1]
    hcat = jnp.concatenate([ps[i] for i in range(L)], axis=1)
    h1 = jnp.dot(hcat.astype(jnp.bfloat16), w1_ref[...].astype(jnp.bfloat16),
                 preferred_element_type=jnp.float32)
    h1 = jnp.maximum(h1 + b1_ref[...], 0.0)
    out = jnp.sum(h1.astype(jnp.bfloat16).astype(jnp.float32)
                  * w2_ref[...].astype(jnp.bfloat16).astype(jnp.float32),
                  axis=1, keepdims=True) + b2_ref[...]
    out_ref[...] = out
    sig_ref[...] = jax.nn.sigmoid(out)


def _tc_readout(pools, lin1_w, lin1_b, lin2_w, lin2_b):
    return pl.pallas_call(
        _tc_readout_body,
        out_shape=(jax.ShapeDtypeStruct((G, 1), jnp.float32),
                   jax.ShapeDtypeStruct((G, 1), jnp.float32)),
    )(pools, lin1_w, lin1_b, lin2_w, lin2_b)


def kernel(x, edge_index, batch, W1, b1, gamma, beta, W2, b2,
           lin1_w, lin1_b, lin2_w, lin2_b):
    src = edge_index[0]
    dst = edge_index[1]
    epad = E_P - E_EDGES
    src_p = jnp.concatenate(
        [src, jnp.zeros((epad,), jnp.int32)]).reshape(E_P // 128, 128)
    dst_p = jnp.concatenate(
        [dst, jnp.full((epad,), N_NODES, jnp.int32)]).reshape(E_P // 128, 128)
    batch_p = jnp.concatenate(
        [batch, jnp.zeros((N_PAD,), jnp.int32)]).reshape(N_CH, 128)
    h = jnp.pad(x, ((0, N_PAD), (0, 0)))

    eye8 = jnp.eye(8, dtype=jnp.float32)
    eye16 = jnp.eye(H, dtype=jnp.float32)
    w1b = jnp.einsum("ab,lij->laibj", eye8, W1).reshape(L, 128, 128)
    w2b = jnp.einsum("ab,lij->laibj", eye8, W2).reshape(L, 128, 128)
    b1t = jnp.tile(b1, (1, 8))
    b2t = jnp.tile(b2, (1, 8))
    betat = jnp.tile(beta, (1, 8))

    pools = []
    for i in range(L):
        aggp, poolp = _sc_agg_pool(h, src_p, dst_p, batch_p)
        if i > 0:
            pools.append(poolp)
        h2 = _tc_layer(h.reshape(N_ROWS, 128), aggp.reshape(2, N_ROWS, 128),
                       w1b[i], b1t[i].reshape(1, 128), b1[i].reshape(1, H),
                       gamma[i].reshape(1, H), betat[i].reshape(1, 128),
                       w2b[i], b2t[i].reshape(1, 128))
        h = h2.reshape(N_P, H)
    pools.append(_sc_pool(h, batch_p))
    allp = jnp.stack(pools)
    sig, out = _tc_readout(allp, lin1_w, lin1_b.reshape(1, L * H),
                           lin2_w.reshape(1, L * H), lin2_b.reshape(1, 1))
    return (sig, out)
